# baseline (speedup 1.0000x reference)
import functools

import jax
import jax.numpy as jnp
from jax import lax
from jax.experimental import pallas as pl
from jax.experimental.pallas import tpu as pltpu
from jax.experimental.pallas import tpu_sc as plsc

L = 16
NC, NS = 2, 16
NW = NC * NS
B, S, D = 4096, 200, 64
DW = 2 * D
VOCAB_HALF = 500000
SP = S // 2
PP = 112
PT = 2 * PP
DG = D // L
RPW = B // NW
RB = 16
NEG = -1e30


def _make_encoder():
    mesh = plsc.VectorSubcoreMesh(core_axis_name="c", subcore_axis_name="s")

    @functools.partial(
        pl.kernel,
        mesh=mesh,
        compiler_params=pltpu.CompilerParams(needs_layout_passes=False),
        out_type=jax.ShapeDtypeStruct((B, D), jnp.float32),
        scratch_types=[
            pltpu.VMEM((RB, 4, SP), jnp.int32),
            pltpu.VMEM((RB, PT), jnp.float32),
            pltpu.VMEM((PT, DW), jnp.float32),
            pltpu.VMEM((PT,), jnp.float32),
            pltpu.VMEM((RB, D), jnp.float32),
            pltpu.SemaphoreType.DMA,
        ],
    )
    def enc(idx_hbm, par_hbm, emb_hbm, w_hbm, out_hbm,
            idx_v, par_v, rows_v, wv, outv, sem):
        cid = lax.axis_index("c")
        sid = lax.axis_index("s")
        wid = sid * NC + cid
        base = wid * RPW

        zv16 = jnp.zeros((L,), dtype=jnp.float32)
        for r0 in (SP, PP + SP):
            for rr in range(PP - SP):
                for g in range(DW // L):
                    rows_v[r0 + rr, pl.ds(g * L, L)] = zv16

        def block_loop(blk, carry0):
            rbase = base + blk * RB
            pltpu.sync_copy(idx_hbm.at[pl.ds(rbase, RB)], idx_v)
            pltpu.sync_copy(par_hbm.at[pl.ds(rbase, RB)], par_v)

            def row_loop(r, carry1):
                pad = jnp.full((L,), NEG, dtype=jnp.float32)
                wv[pl.ds(SP - 4, L)] = pad
                wv[pl.ds(PP + SP - 4, L)] = pad
                c0 = pltpu.async_copy(
                    emb_hbm.at[idx_v.at[r, 0]], rows_v.at[pl.ds(0, SP)], sem)
                c1 = pltpu.async_copy(
                    emb_hbm.at[idx_v.at[r, 1]], rows_v.at[pl.ds(PP, SP)], sem)
                c2 = pltpu.async_copy(
                    w_hbm.at[idx_v.at[r, 2]], wv.at[pl.ds(0, SP)], sem)
                c3 = pltpu.async_copy(
                    w_hbm.at[idx_v.at[r, 3]], wv.at[pl.ds(PP, SP)], sem)
                c0.wait()
                c1.wait()
                c2.wait()
                c3.wait()

                m = jnp.full((L,), NEG, dtype=jnp.float32)
                for c in range(PT // L):
                    m = jnp.maximum(m, wv[pl.ds(c * L, L)])
                mm = jnp.max(m)
                zsum = jnp.zeros((L,), dtype=jnp.float32)
                for c in range(PT // L):
                    e = jnp.exp(wv[pl.ds(c * L, L)] - mm)
                    wv[pl.ds(c * L, L)] = e
                    zsum = zsum + e
                rzv = jnp.ones((L,), dtype=jnp.float32) / jnp.broadcast_to(
                    jnp.sum(zsum), (L,))

                def grp_body(g, acc):
                    wg = wv[pl.ds(g * L, L)]
                    pg = par_v[r, pl.ds(g * L, L)]
                    row0 = g * L
                    for k in range(L):
                        wk = wg[k]
                        pk = pg[k]
                        we = wk - wk * pk
                        wo = wk * pk
                        acc = tuple(
                            acc[d]
                            + we * rows_v[row0 + k, pl.ds(d * L, L)]
                            + wo * rows_v[row0 + k, pl.ds(D + d * L, L)]
                            for d in range(DG))
                    return acc

                accs = lax.fori_loop(
                    0, PT // L, grp_body,
                    tuple(jnp.zeros((L,), dtype=jnp.float32)
                          for _ in range(DG)))
                for d in range(DG):
                    outv[r, pl.ds(d * L, L)] = accs[d] * rzv
                return carry1

            lax.fori_loop(0, RB, row_loop, 0)
            pltpu.sync_copy(outv, out_hbm.at[pl.ds(rbase, RB)])
            return carry0

        lax.fori_loop(0, RPW // RB, block_loop, 0)

    return enc


_ENCODER = _make_encoder()


def kernel(document, lens, embed_table, weight_table):
    del lens
    doc = document.astype(jnp.int32)
    half3 = (doc >> 1).reshape(B, 2, SP)
    orig3 = doc.reshape(B, 2, SP)
    idx = jnp.concatenate([half3, orig3], axis=1)
    par2 = (doc & 1).astype(jnp.float32).reshape(B, 2, SP)
    par = jnp.zeros((B, 2, PP), dtype=jnp.float32)
    par = par.at[:, :, :SP].set(par2).reshape(B, PT)
    emb2 = embed_table.reshape(VOCAB_HALF, DW)
    wflat = weight_table.reshape(-1)
    return _ENCODER(idx, par, emb2, wflat)
